# baseline (speedup 1.0000x reference)
import functools

import jax
import jax.numpy as jnp
from jax import lax
from jax.experimental import pallas as pl
from jax.experimental.pallas import tpu as pltpu
from jax.experimental.pallas import tpu_sc as plsc

F32 = jnp.float32
I32 = jnp.int32

NTILE = 16
CH = 128
SCH = 16
NBUF = 4
NSC = 2
WB = 128


def _mm_body(xu, xv, wu, wv, tu0, tu1, tv0, tv1):
    w0u = wu[0]
    w1u = wu[0] + wu[1]
    w0v = wv[0]
    w1v = wv[0] + wv[1]
    a = xu[...]
    b = xv[...]
    tu0[...] = jnp.dot(a, w0u, preferred_element_type=F32)
    tu1[...] = jnp.dot(a, w1u, preferred_element_type=F32)
    tv0[...] = jnp.dot(b, w0v, preferred_element_type=F32)
    tv1[...] = jnp.dot(b, w1v, preferred_element_type=F32)


def _project(x_u, x_v, weights_u, weights_v):
    n, d = x_u.shape
    h = weights_u.shape[2]
    blk = 2000
    grid = n // blk
    return pl.pallas_call(
        _mm_body,
        grid=(grid,),
        in_specs=[
            pl.BlockSpec((blk, d), lambda i: (i, 0)),
            pl.BlockSpec((blk, d), lambda i: (i, 0)),
            pl.BlockSpec(weights_u.shape, lambda i: (0, 0, 0)),
            pl.BlockSpec(weights_v.shape, lambda i: (0, 0, 0)),
        ],
        out_specs=[pl.BlockSpec((blk, h), lambda i: (i, 0))] * 4,
        out_shape=[jax.ShapeDtypeStruct((n, h), F32)] * 4,
    )(x_u, x_v, weights_u, weights_v)


def _make_sc_agg(rows_pad, half, chunks):
    mesh = plsc.VectorSubcoreMesh(core_axis_name="c", subcore_axis_name="s")
    rows_per_tile = rows_pad // NTILE
    wblocks = rows_per_tile // WB
    hw = half // 2

    @functools.partial(
        pl.kernel,
        out_type=jax.ShapeDtypeStruct((2, rows_pad, half), F32),
        mesh=mesh,
        scratch_types=[
            pltpu.VMEM((SCH, CH), I32),
            pltpu.VMEM((SCH, CH), I32),
            pltpu.VMEM((SCH, CH), F32),
            [pltpu.VMEM((CH, hw), I32)] * NBUF,
            [pltpu.VMEM((CH, half), F32)] * NSC,
            [pltpu.SemaphoreType.DMA] * NBUF,
            [pltpu.SemaphoreType.DMA] * NSC,
            pltpu.VMEM_SHARED((rows_pad, half), F32),
        ],
        compiler_params=pltpu.CompilerParams(use_tc_tiling_on_sc=False),
    )
    def agg(tabs, gidx, sidx, vl,
            out, gbuf, sbuf, vbuf, Gb, G2, semg, sems, acc):
        cid = lax.axis_index("c")
        sid = lax.axis_index("s")
        wbuf = G2[0]

        def zero_body(k, _):
            wbuf[k // 4, pl.ds((k % 4) * 16, 16)] = jnp.zeros((16,), F32)
            return 0
        lax.fori_loop(0, WB * half // 16, zero_body, 0)
        for t in range(wblocks):
            pltpu.sync_copy(wbuf, acc.at[pl.ds(sid * rows_per_tile + t * WB, WB)])
        plsc.subcore_barrier()

        def scale_chunk(b, c, j):
            @plsc.parallel_loop(0, CH // 16, unroll=1)
            def group_body(g):
                ev = vbuf[j, pl.ds(g * 16, 16)]
                for l in range(16):
                    spl = jnp.full((16,), ev[l], F32)
                    e2 = g * 16 + l
                    for h2 in range(hw // 16):
                        v = Gb[b][e2, pl.ds(h2 * 16, 16)]
                        lo = lax.bitcast_convert_type(v << 16, F32)
                        hi = lax.bitcast_convert_type(v & jnp.int32(-65536), F32)
                        G2[c][e2, pl.ds(h2 * 16, 16)] = lo * spl
                        G2[c][e2, pl.ds(hw + h2 * 16, 16)] = hi * spl

        def sup_body(i, _):
            tab = tabs.at[cid, i]

            def seg_body(s0, _):
                pltpu.sync_copy(gidx.at[cid, i, sid, pl.ds(s0 * SCH, SCH)], gbuf)
                pltpu.sync_copy(sidx.at[cid, i, sid, pl.ds(s0 * SCH, SCH)], sbuf)
                pltpu.sync_copy(vl.at[i, sid, pl.ds(s0 * SCH, SCH)], vbuf)
                for p in range(NBUF - 1):
                    pltpu.async_copy(tab.at[gbuf.at[p]], Gb[p], semg[p])

                def quad_body(q, _):
                    for b in range(NBUF):
                        j = q * NBUF + b
                        c = b % NSC
                        nb = (b + NBUF - 1) % NBUF

                        def prefetch():
                            pltpu.async_copy(
                                tab.at[gbuf.at[j + NBUF - 1]], Gb[nb],
                                semg[nb])
                        pl.when(j + NBUF - 1 < SCH)(prefetch)

                        pltpu.make_async_copy(
                            tab.at[gbuf.at[j]], Gb[b], semg[b]).wait()

                        scale_chunk(b, c, j)
                    return 0
                lax.fori_loop(0, SCH // NBUF, quad_body, 0)
                return 0
            lax.fori_loop(0, chunks // SCH, seg_body, 0)
            return 0
        lax.fori_loop(0, 2, sup_body, 0)
        plsc.subcore_barrier()

        for t in range(wblocks):
            r0 = sid * rows_per_tile + t * WB
            pltpu.sync_copy(acc.at[pl.ds(r0, WB)], wbuf)

            def relu_body(k, _):
                sl = pl.ds((k % 4) * 16, 16)
                wbuf[k // 4, sl] = jnp.maximum(wbuf[k // 4, sl], 0.0)
                return 0
            lax.fori_loop(0, WB * half // 16, relu_body, 0)
            pltpu.sync_copy(wbuf, out.at[cid, pl.ds(r0, WB)])

    return agg


def kernel(x_u, x_v, edge_index_0, edge_index_1, edge_val_0, edge_val_1,
           weights_u, weights_v):
    nu = x_u.shape[0]
    nv = x_v.shape[0]
    half = weights_u.shape[2]
    e = edge_index_0.shape[1]

    per_tile = -(-e // NTILE)
    chunks = -(-per_tile // CH)
    chunks = -(-chunks // SCH) * SCH
    e_pad = NTILE * chunks * CH

    tu0, tu1, tv0, tv1 = _project(x_u, x_v, weights_u, weights_v)

    def pack_tab(t):
        n = t.shape[0]
        sw = t.reshape(n, 2, half // 2).transpose(0, 2, 1)
        return lax.bitcast_convert_type(sw.astype(jnp.bfloat16), I32)

    ptu0, ptu1, ptv0, ptv1 = (pack_tab(t) for t in (tu0, tu1, tv0, tv1))

    spread = jnp.arange(e_pad - e, dtype=I32) % nu

    def rs(a, padv):
        return jnp.concatenate([a, padv]).reshape(NTILE, chunks, CH)

    ei0 = edge_index_0.astype(I32)
    ei1 = edge_index_1.astype(I32)
    row0, col0 = ei0[0], ei0[1]
    row1, col1 = ei1[0], ei1[1]

    gu = jnp.stack([rs(col0, spread), rs(col1, spread)])
    su = jnp.stack([rs(2 * row0, 2 * spread), rs(2 * row1 + 1, 2 * spread)])
    gv = jnp.stack([rs(row0, spread), rs(row1, spread)])
    sv = jnp.stack([rs(2 * col0, 2 * spread), rs(2 * col1 + 1, 2 * spread)])
    zpad = jnp.zeros(e_pad - e, F32)
    vl = jnp.stack([rs(edge_val_0.astype(F32), zpad),
                    rs(edge_val_1.astype(F32), zpad)])

    tabs = jnp.stack([jnp.stack([ptv0, ptv1]), jnp.stack([ptu0, ptu1])])
    gidx = jnp.stack([gu, gv])
    sidx = jnp.stack([su, sv])

    blk = NTILE * WB
    rows_pad = -(-2 * nu // blk) * blk
    agg = _make_sc_agg(rows_pad, half, chunks)
    out = agg(tabs, gidx, sidx, vl)
    return (out[0, :2 * nu].reshape(nu, 2 * half),
            out[1, :2 * nv].reshape(nv, 2 * half))


# speedup vs baseline: 1.2552x; 1.2552x over previous
import functools

import jax
import jax.numpy as jnp
from jax import lax
from jax.experimental import pallas as pl
from jax.experimental.pallas import tpu as pltpu
from jax.experimental.pallas import tpu_sc as plsc

F32 = jnp.float32
I32 = jnp.int32

NTILE = 16
CH = 128
SCH = 16
NBUF = 4
WB = 128


def _mm_body(xu, xv, wu, wv, tu0, tu1, tv0, tv1):
    w0u = wu[0]
    w1u = wu[0] + wu[1]
    w0v = wv[0]
    w1v = wv[0] + wv[1]
    a = xu[...]
    b = xv[...]
    tu0[...] = jnp.dot(a, w0u, preferred_element_type=F32)
    tu1[...] = jnp.dot(a, w1u, preferred_element_type=F32)
    tv0[...] = jnp.dot(b, w0v, preferred_element_type=F32)
    tv1[...] = jnp.dot(b, w1v, preferred_element_type=F32)


def _project(x_u, x_v, weights_u, weights_v):
    n, d = x_u.shape
    h = weights_u.shape[2]
    blk = 2000
    grid = n // blk
    return pl.pallas_call(
        _mm_body,
        grid=(grid,),
        in_specs=[
            pl.BlockSpec((blk, d), lambda i: (i, 0)),
            pl.BlockSpec((blk, d), lambda i: (i, 0)),
            pl.BlockSpec(weights_u.shape, lambda i: (0, 0, 0)),
            pl.BlockSpec(weights_v.shape, lambda i: (0, 0, 0)),
        ],
        out_specs=[pl.BlockSpec((blk, h), lambda i: (i, 0))] * 4,
        out_shape=[jax.ShapeDtypeStruct((n, h), F32)] * 4,
    )(x_u, x_v, weights_u, weights_v)


def _make_sc_agg(rows_pad, half, chunks):
    mesh = plsc.VectorSubcoreMesh(core_axis_name="c", subcore_axis_name="s")
    rows_per_tile = rows_pad // NTILE
    wblocks = rows_per_tile // WB

    @functools.partial(
        pl.kernel,
        out_type=jax.ShapeDtypeStruct((2, rows_pad, 2 * half), F32),
        mesh=mesh,
        scratch_types=[
            pltpu.VMEM((SCH, CH), I32),
            pltpu.VMEM((SCH, CH), I32),
            pltpu.VMEM((SCH, CH), F32),
            [pltpu.VMEM((CH, half), F32)] * NBUF,
            [pltpu.SemaphoreType.DMA] * NBUF,
            [pltpu.SemaphoreType.DMA] * NBUF,
            pltpu.VMEM_SHARED((rows_pad, half), F32),
            pltpu.VMEM_SHARED((rows_pad, half), F32),
        ],
        compiler_params=pltpu.CompilerParams(use_tc_tiling_on_sc=False),
    )
    def agg(tabs, gidx, sidx, vl,
            out, gbuf, sbuf, vbuf, G, semg, sems, stab, acc):
        cid = lax.axis_index("c")
        sid = lax.axis_index("s")
        wbuf = G[0]
        r_tile = sid * rows_per_tile

        def scale_chunk(b, j):
            @plsc.parallel_loop(0, CH // 16, unroll=2)
            def group_body(g):
                ev = vbuf[j, pl.ds(g * 16, 16)]
                for l in range(16):
                    spl = jnp.full((16,), ev[l], F32)
                    e2 = g * 16 + l
                    for q in range(half // 16):
                        sl = pl.ds(q * 16, 16)
                        G[b][e2, sl] = G[b][e2, sl] * spl

        def sup_body(i, _):
            pltpu.sync_copy(tabs.at[cid, i, pl.ds(r_tile, rows_per_tile)],
                            stab.at[pl.ds(r_tile, rows_per_tile)])

            def zero_body(k, _):
                wbuf[k // 4, pl.ds((k % 4) * 16, 16)] = jnp.zeros((16,), F32)
                return 0
            lax.fori_loop(0, WB * half // 16, zero_body, 0)
            for t in range(wblocks):
                pltpu.sync_copy(wbuf, acc.at[pl.ds(r_tile + t * WB, WB)])
            plsc.subcore_barrier()

            def seg_body(s0, _):
                pltpu.sync_copy(gidx.at[cid, i, sid, pl.ds(s0 * SCH, SCH)], gbuf)
                pltpu.sync_copy(sidx.at[cid, i, sid, pl.ds(s0 * SCH, SCH)], sbuf)
                pltpu.sync_copy(vl.at[i, sid, pl.ds(s0 * SCH, SCH)], vbuf)
                for p in range(2):
                    pltpu.async_copy(stab.at[gbuf.at[p]], G[p], semg[p])

                def quad_body(q, _):
                    for b in range(NBUF):
                        j = q * NBUF + b
                        nb = (b + 2) % NBUF

                        def prefetch():
                            def drain_prev():
                                pltpu.make_async_copy(
                                    G[nb], acc.at[sbuf.at[j - 2]], sems[nb]
                                ).wait()
                            pl.when(j >= 2)(drain_prev)
                            pltpu.async_copy(
                                stab.at[gbuf.at[j + 2]], G[nb], semg[nb])
                        pl.when(j + 2 < SCH)(prefetch)

                        pltpu.make_async_copy(
                            stab.at[gbuf.at[j]], G[b], semg[b]).wait()
                        scale_chunk(b, j)
                        pltpu.async_copy(
                            G[b], acc.at[sbuf.at[j]], sems[b], add=True)
                    return 0
                lax.fori_loop(0, SCH // NBUF, quad_body, 0)

                for b in range(NBUF):
                    pltpu.make_async_copy(
                        G[b], acc.at[sbuf.at[SCH - NBUF + b]], sems[b]
                    ).wait()
                return 0
            lax.fori_loop(0, chunks // SCH, seg_body, 0)
            plsc.subcore_barrier()

            col0 = pl.multiple_of(i * half, 8)
            for t in range(wblocks):
                r0 = r_tile + t * WB
                pltpu.sync_copy(acc.at[pl.ds(r0, WB)], wbuf)

                def relu_body(k, _):
                    sl = pl.ds((k % 4) * 16, 16)
                    wbuf[k // 4, sl] = jnp.maximum(wbuf[k // 4, sl], 0.0)
                    return 0
                lax.fori_loop(0, WB * half // 16, relu_body, 0)
                pltpu.sync_copy(wbuf,
                                out.at[cid, pl.ds(r0, WB), pl.ds(col0, half)])
            return 0
        lax.fori_loop(0, 2, sup_body, 0)

    return agg


def kernel(x_u, x_v, edge_index_0, edge_index_1, edge_val_0, edge_val_1,
           weights_u, weights_v):
    nu = x_u.shape[0]
    nv = x_v.shape[0]
    half = weights_u.shape[2]
    e = edge_index_0.shape[1]

    per_tile = -(-e // NTILE)
    chunks = -(-per_tile // CH)
    chunks = -(-chunks // SCH) * SCH
    e_pad = NTILE * chunks * CH

    blk = NTILE * WB
    rows_pad = -(-nu // blk) * blk

    tu0, tu1, tv0, tv1 = _project(x_u, x_v, weights_u, weights_v)

    def padr(t):
        return jnp.pad(t, ((0, rows_pad - t.shape[0]), (0, 0)))

    tabs = jnp.stack([jnp.stack([padr(tv0), padr(tv1)]),
                      jnp.stack([padr(tu0), padr(tu1)])])

    spread = jnp.arange(e_pad - e, dtype=I32) % nu

    def rs(a, padv):
        return jnp.concatenate([a, padv]).reshape(NTILE, chunks, CH)

    ei0 = edge_index_0.astype(I32)
    ei1 = edge_index_1.astype(I32)
    row0, col0 = ei0[0], ei0[1]
    row1, col1 = ei1[0], ei1[1]

    gidx = jnp.stack([jnp.stack([rs(col0, spread), rs(col1, spread)]),
                      jnp.stack([rs(row0, spread), rs(row1, spread)])])
    sidx = jnp.stack([jnp.stack([rs(row0, spread), rs(row1, spread)]),
                      jnp.stack([rs(col0, spread), rs(col1, spread)])])
    zpad = jnp.zeros(e_pad - e, F32)
    vl = jnp.stack([rs(edge_val_0.astype(F32), zpad),
                    rs(edge_val_1.astype(F32), zpad)])

    agg = _make_sc_agg(rows_pad, half, chunks)
    out = agg(tabs, gidx, sidx, vl)
    return (out[0, :nu], out[1, :nv])


# speedup vs baseline: 1.3684x; 1.0901x over previous
import functools

import jax
import jax.numpy as jnp
from jax import lax
from jax.experimental import pallas as pl
from jax.experimental.pallas import tpu as pltpu
from jax.experimental.pallas import tpu_sc as plsc

F32 = jnp.float32
I32 = jnp.int32

NTILE = 16
CH = 128
SCH = 16
NBUF = 4
WB = 128


def _mm_body(xu, xv, wu, wv, tu0, tu1, tv0, tv1):
    w0u = wu[0]
    w1u = wu[0] + wu[1]
    w0v = wv[0]
    w1v = wv[0] + wv[1]
    a = xu[...]
    b = xv[...]
    tu0[...] = jnp.dot(a, w0u, preferred_element_type=F32)
    tu1[...] = jnp.dot(a, w1u, preferred_element_type=F32)
    tv0[...] = jnp.dot(b, w0v, preferred_element_type=F32)
    tv1[...] = jnp.dot(b, w1v, preferred_element_type=F32)


def _project(x_u, x_v, weights_u, weights_v):
    n, d = x_u.shape
    h = weights_u.shape[2]
    blk = 2000
    grid = n // blk
    return pl.pallas_call(
        _mm_body,
        grid=(grid,),
        in_specs=[
            pl.BlockSpec((blk, d), lambda i: (i, 0)),
            pl.BlockSpec((blk, d), lambda i: (i, 0)),
            pl.BlockSpec(weights_u.shape, lambda i: (0, 0, 0)),
            pl.BlockSpec(weights_v.shape, lambda i: (0, 0, 0)),
        ],
        out_specs=[pl.BlockSpec((blk, h), lambda i: (i, 0))] * 4,
        out_shape=[jax.ShapeDtypeStruct((n, h), F32)] * 4,
    )(x_u, x_v, weights_u, weights_v)


def _make_sc_agg(rows_pad, half, chunks):
    mesh = plsc.VectorSubcoreMesh(core_axis_name="c", subcore_axis_name="s")
    rows_per_tile = rows_pad // NTILE
    wblocks = rows_per_tile // WB

    nseg = chunks // SCH
    assert nseg % 2 == 0

    @functools.partial(
        pl.kernel,
        out_type=jax.ShapeDtypeStruct((2, rows_pad, 2 * half), F32),
        mesh=mesh,
        scratch_types=[
            [pltpu.VMEM((SCH, 3, CH), I32)] * 2,
            [pltpu.VMEM((CH, half), F32)] * NBUF,
            [pltpu.SemaphoreType.DMA] * 2,
            [pltpu.SemaphoreType.DMA] * NBUF,
            [pltpu.SemaphoreType.DMA] * NBUF,
            pltpu.VMEM_SHARED((rows_pad, half), F32),
            pltpu.VMEM_SHARED((rows_pad, half), F32),
        ],
        compiler_params=pltpu.CompilerParams(use_tc_tiling_on_sc=False),
    )
    def agg(tabs, allidx,
            out, ibuf, G, semi, semg, sems, stab, acc):
        cid = lax.axis_index("c")
        sid = lax.axis_index("s")
        wbuf = G[0]
        r_tile = sid * rows_per_tile

        def scale_chunk(b, j, par):
            @plsc.parallel_loop(0, CH // 16, unroll=2)
            def group_body(g):
                ev = lax.bitcast_convert_type(
                    ibuf[par][j, 2, pl.ds(g * 16, 16)], F32)
                for l in range(16):
                    spl = jnp.full((16,), ev[l], F32)
                    e2 = g * 16 + l
                    for q in range(half // 16):
                        sl = pl.ds(q * 16, 16)
                        G[b][e2, sl] = G[b][e2, sl] * spl

        def sup_body(i, _):
            pltpu.sync_copy(tabs.at[cid, i, pl.ds(r_tile, rows_per_tile)],
                            stab.at[pl.ds(r_tile, rows_per_tile)])

            def zero_body(k, _):
                wbuf[k // 4, pl.ds((k % 4) * 16, 16)] = jnp.zeros((16,), F32)
                return 0
            lax.fori_loop(0, WB * half // 16, zero_body, 0)
            for t in range(wblocks):
                pltpu.sync_copy(wbuf, acc.at[pl.ds(r_tile + t * WB, WB)])
            plsc.subcore_barrier()

            pltpu.sync_copy(allidx.at[cid, i, sid, pl.ds(0, SCH)], ibuf[0])
            for p2 in range(2):
                pltpu.async_copy(stab.at[ibuf[0].at[p2, 0]], G[p2], semg[p2])

            def pair_body(sp, _):
                for par in range(2):
                    s0 = sp * 2 + par
                    ib = ibuf[par]
                    nxt = ibuf[1 - par]
                    has_next = s0 + 1 < nseg

                    def quad_body(q, _):
                        for b in range(NBUF):
                            j = q * NBUF + b
                            nb = (b + 2) % NBUF

                            def stage_next():
                                pltpu.async_copy(
                                    allidx.at[cid, i, sid,
                                              pl.ds((s0 + 1) * SCH, SCH)],
                                    nxt, semi[1 - par])
                            pl.when((j == 2) & has_next)(stage_next)

                            def wait_stage():
                                pltpu.make_async_copy(
                                    allidx.at[cid, i, sid, pl.ds(0, SCH)],
                                    nxt, semi[1 - par]).wait()
                            pl.when((j == SCH - 2) & has_next)(wait_stage)

                            def drain_nb():
                                pltpu.make_async_copy(
                                    G[nb], acc.at[ib.at[0, 1]], sems[nb]
                                ).wait()

                            def pref_in():
                                def drain_a():
                                    pl.when((s0 > 0) | (j >= 2))(drain_nb)
                                drain_a()
                                pltpu.async_copy(
                                    stab.at[ib.at[j + 2, 0]], G[nb], semg[nb])
                            pl.when(j + 2 < SCH)(pref_in)

                            def pref_cross():
                                drain_nb()
                                pltpu.async_copy(
                                    stab.at[nxt.at[j + 2 - SCH, 0]], G[nb],
                                    semg[nb])
                            pl.when((j + 2 >= SCH) & has_next)(pref_cross)

                            pltpu.make_async_copy(
                                stab.at[ib.at[j, 0]], G[b], semg[b]).wait()
                            scale_chunk(b, j, par)
                            pltpu.async_copy(
                                G[b], acc.at[ib.at[j, 1]], sems[b], add=True)
                        return 0
                    lax.fori_loop(0, SCH // NBUF, quad_body, 0)
                return 0
            lax.fori_loop(0, nseg // 2, pair_body, 0)

            for b in range(NBUF):
                pltpu.make_async_copy(
                    G[b], acc.at[ibuf[0].at[0, 1]], sems[b]).wait()
            plsc.subcore_barrier()

            col0 = pl.multiple_of(i * half, 8)
            for t in range(wblocks):
                r0 = r_tile + t * WB
                pltpu.sync_copy(acc.at[pl.ds(r0, WB)], wbuf)

                def relu_body(k, _):
                    sl = pl.ds((k % 4) * 16, 16)
                    wbuf[k // 4, sl] = jnp.maximum(wbuf[k // 4, sl], 0.0)
                    return 0
                lax.fori_loop(0, WB * half // 16, relu_body, 0)
                pltpu.sync_copy(wbuf,
                                out.at[cid, pl.ds(r0, WB), pl.ds(col0, half)])
            return 0
        lax.fori_loop(0, 2, sup_body, 0)

    return agg


def kernel(x_u, x_v, edge_index_0, edge_index_1, edge_val_0, edge_val_1,
           weights_u, weights_v):
    nu = x_u.shape[0]
    nv = x_v.shape[0]
    half = weights_u.shape[2]
    e = edge_index_0.shape[1]

    per_tile = -(-e // NTILE)
    chunks = -(-per_tile // CH)
    chunks = -(-chunks // SCH) * SCH
    e_pad = NTILE * chunks * CH

    blk = NTILE * WB
    rows_pad = -(-nu // blk) * blk

    tu0, tu1, tv0, tv1 = _project(x_u, x_v, weights_u, weights_v)

    def padr(t):
        return jnp.pad(t, ((0, rows_pad - t.shape[0]), (0, 0)))

    tabs = jnp.stack([jnp.stack([padr(tv0), padr(tv1)]),
                      jnp.stack([padr(tu0), padr(tu1)])])

    spread = jnp.arange(e_pad - e, dtype=I32) % nu

    def rs(a, padv):
        return jnp.concatenate([a, padv]).reshape(NTILE, chunks, CH)

    ei0 = edge_index_0.astype(I32)
    ei1 = edge_index_1.astype(I32)
    row0, col0 = ei0[0], ei0[1]
    row1, col1 = ei1[0], ei1[1]

    zpad = jnp.zeros(e_pad - e, I32)
    v0 = lax.bitcast_convert_type(edge_val_0.astype(F32), I32)
    v1 = lax.bitcast_convert_type(edge_val_1.astype(F32), I32)

    def sup_pack(g, s, v):
        return jnp.stack([g, s, v], axis=2)

    allidx = jnp.stack([
        jnp.stack([sup_pack(rs(col0, spread), rs(row0, spread), rs(v0, zpad)),
                   sup_pack(rs(col1, spread), rs(row1, spread), rs(v1, zpad))]),
        jnp.stack([sup_pack(rs(row0, spread), rs(col0, spread), rs(v0, zpad)),
                   sup_pack(rs(row1, spread), rs(col1, spread), rs(v1, zpad))]),
    ])

    agg = _make_sc_agg(rows_pad, half, chunks)
    out = agg(tabs, allidx)
    return (out[0, :nu], out[1, :nv])


# speedup vs baseline: 1.5356x; 1.1222x over previous
import functools

import jax
import jax.numpy as jnp
from jax import lax
from jax.experimental import pallas as pl
from jax.experimental.pallas import tpu as pltpu
from jax.experimental.pallas import tpu_sc as plsc

F32 = jnp.float32
I32 = jnp.int32

NTILE = 16
CH = 128
SCH = 16
NBUF = 4
WB = 128


def _mm_body(xu, xv, wu, wv, tu0, tu1, tv0, tv1):
    w0u = wu[0]
    w1u = wu[0] + wu[1]
    w0v = wv[0]
    w1v = wv[0] + wv[1]
    a = xu[...]
    b = xv[...]
    tu0[...] = jnp.dot(a, w0u, preferred_element_type=F32)
    tu1[...] = jnp.dot(a, w1u, preferred_element_type=F32)
    tv0[...] = jnp.dot(b, w0v, preferred_element_type=F32)
    tv1[...] = jnp.dot(b, w1v, preferred_element_type=F32)


def _project(x_u, x_v, weights_u, weights_v):
    n, d = x_u.shape
    h = weights_u.shape[2]
    blk = 2000
    grid = n // blk
    return pl.pallas_call(
        _mm_body,
        grid=(grid,),
        in_specs=[
            pl.BlockSpec((blk, d), lambda i: (i, 0)),
            pl.BlockSpec((blk, d), lambda i: (i, 0)),
            pl.BlockSpec(weights_u.shape, lambda i: (0, 0, 0)),
            pl.BlockSpec(weights_v.shape, lambda i: (0, 0, 0)),
        ],
        out_specs=[pl.BlockSpec((blk, h), lambda i: (i, 0))] * 4,
        out_shape=[jax.ShapeDtypeStruct((n, h), F32)] * 4,
    )(x_u, x_v, weights_u, weights_v)


def _make_sc_agg(rows_pad, half, chunks):
    mesh = plsc.VectorSubcoreMesh(core_axis_name="c", subcore_axis_name="s")
    rows_per_tile = rows_pad // NTILE
    wblocks = rows_per_tile // WB

    nseg = chunks // SCH
    assert nseg % 2 == 0

    @functools.partial(
        pl.kernel,
        out_type=jax.ShapeDtypeStruct((2, rows_pad, 2 * half), F32),
        mesh=mesh,
        scratch_types=[
            [pltpu.VMEM((SCH, 2, CH), I32)] * 2,
            [pltpu.VMEM((SCH, CH), F32)] * 2,
            [pltpu.VMEM((CH, half), F32)] * NBUF,
            [pltpu.SemaphoreType.DMA] * 2,
            [pltpu.SemaphoreType.DMA] * NBUF,
            [pltpu.SemaphoreType.DMA] * NBUF,
            pltpu.SemaphoreType.DMA,
            pltpu.VMEM_SHARED((rows_pad, half), F32),
            pltpu.VMEM_SHARED((rows_pad, half), F32),
        ],
        compiler_params=pltpu.CompilerParams(use_tc_tiling_on_sc=False),
    )
    def agg(tabs, rc, vl,
            out, ibuf, vbuf, G, semi, semg, sems, semt, stab, acc):
        cid = lax.axis_index("c")
        sid = lax.axis_index("s")
        gsel = 1 - cid
        ssel = cid
        wbuf = G[0]
        zbuf = G[1]
        r_tile = sid * rows_per_tile

        def scale_chunk(b, j, par):
            @plsc.parallel_loop(0, CH // 16, unroll=2)
            def group_body(g):
                ev = vbuf[par][j, pl.ds(g * 16, 16)]
                for l in range(16):
                    spl = jnp.full((16,), ev[l], F32)
                    e2 = g * 16 + l
                    for q in range(half // 16):
                        sl = pl.ds(q * 16, 16)
                        G[b][e2, sl] = G[b][e2, sl] * spl

        def zero_body(k, _):
            zbuf[k // 4, pl.ds((k % 4) * 16, 16)] = jnp.zeros((16,), F32)
            return 0
        lax.fori_loop(0, WB * half // 16, zero_body, 0)

        def sup_body(i, _):
            def first_setup():
                pltpu.sync_copy(tabs.at[cid, 0, pl.ds(r_tile, rows_per_tile)],
                                stab.at[pl.ds(r_tile, rows_per_tile)])
                for t in range(wblocks):
                    pltpu.sync_copy(zbuf, acc.at[pl.ds(r_tile + t * WB, WB)])

            def next_setup():
                pltpu.make_async_copy(
                    tabs.at[cid, 1, pl.ds(r_tile, rows_per_tile)],
                    stab.at[pl.ds(r_tile, rows_per_tile)], semt).wait()
            pl.when(i == 0)(first_setup)
            pl.when(i == 1)(next_setup)
            plsc.subcore_barrier()

            pltpu.sync_copy(rc.at[i, sid, pl.ds(0, SCH)], ibuf[0])
            pltpu.sync_copy(vl.at[i, sid, pl.ds(0, SCH)], vbuf[0])
            for p2 in range(2):
                pltpu.async_copy(stab.at[ibuf[0].at[p2, gsel]], G[p2], semg[p2])

            def pair_body(sp, _):
                for par in range(2):
                    s0 = sp * 2 + par
                    ib = ibuf[par]
                    nxt = ibuf[1 - par]
                    has_next = s0 + 1 < nseg

                    def quad_body(q, _):
                        for b in range(NBUF):
                            j = q * NBUF + b
                            nb = (b + 2) % NBUF

                            def stage_next():
                                sl2 = pl.ds((s0 + 1) * SCH, SCH)
                                pltpu.async_copy(
                                    rc.at[i, sid, sl2], nxt, semi[1 - par])
                                pltpu.async_copy(
                                    vl.at[i, sid, sl2], vbuf[1 - par],
                                    semi[1 - par])
                            pl.when((j == 2) & has_next)(stage_next)

                            def wait_stage():
                                pltpu.make_async_copy(
                                    rc.at[i, sid, pl.ds(0, SCH)],
                                    nxt, semi[1 - par]).wait()
                                pltpu.make_async_copy(
                                    vl.at[i, sid, pl.ds(0, SCH)],
                                    vbuf[1 - par], semi[1 - par]).wait()
                            pl.when((j == SCH - 2) & has_next)(wait_stage)

                            def drain_nb():
                                pltpu.make_async_copy(
                                    G[nb], acc.at[ib.at[0, ssel]], sems[nb]
                                ).wait()

                            def pref_in():
                                def drain_a():
                                    pl.when((s0 > 0) | (j >= 2))(drain_nb)
                                drain_a()
                                pltpu.async_copy(
                                    stab.at[ib.at[j + 2, gsel]], G[nb],
                                    semg[nb])
                            pl.when(j + 2 < SCH)(pref_in)

                            def pref_cross():
                                drain_nb()
                                pltpu.async_copy(
                                    stab.at[nxt.at[j + 2 - SCH, gsel]], G[nb],
                                    semg[nb])
                            pl.when((j + 2 >= SCH) & has_next)(pref_cross)

                            pltpu.make_async_copy(
                                stab.at[ib.at[j, gsel]], G[b], semg[b]).wait()
                            scale_chunk(b, j, par)
                            pltpu.async_copy(
                                G[b], acc.at[ib.at[j, ssel]], sems[b],
                                add=True)
                        return 0
                    lax.fori_loop(0, SCH // NBUF, quad_body, 0)
                return 0
            lax.fori_loop(0, nseg // 2, pair_body, 0)

            for b in range(NBUF):
                pltpu.make_async_copy(
                    G[b], acc.at[ibuf[0].at[0, ssel]], sems[b]).wait()
            plsc.subcore_barrier()

            def prefetch_tab():
                pltpu.async_copy(
                    tabs.at[cid, 1, pl.ds(r_tile, rows_per_tile)],
                    stab.at[pl.ds(r_tile, rows_per_tile)], semt)
                lax.fori_loop(0, WB * half // 16, zero_body, 0)
            pl.when(i == 0)(prefetch_tab)

            col0 = pl.multiple_of(i * half, 8)
            for t in range(wblocks):
                r0 = r_tile + t * WB
                pltpu.sync_copy(acc.at[pl.ds(r0, WB)], wbuf)

                def rezero():
                    pltpu.sync_copy(zbuf, acc.at[pl.ds(r0, WB)])
                pl.when(i == 0)(rezero)

                def relu_body(k, _):
                    sl = pl.ds((k % 4) * 16, 16)
                    wbuf[k // 4, sl] = jnp.maximum(wbuf[k // 4, sl], 0.0)
                    return 0
                lax.fori_loop(0, WB * half // 16, relu_body, 0)
                pltpu.sync_copy(wbuf,
                                out.at[cid, pl.ds(r0, WB), pl.ds(col0, half)])
            return 0
        lax.fori_loop(0, 2, sup_body, 0)

    return agg


def kernel(x_u, x_v, edge_index_0, edge_index_1, edge_val_0, edge_val_1,
           weights_u, weights_v):
    nu = x_u.shape[0]
    nv = x_v.shape[0]
    half = weights_u.shape[2]
    e = edge_index_0.shape[1]

    per_tile = -(-e // NTILE)
    chunks = -(-per_tile // CH)
    chunks = -(-chunks // SCH) * SCH
    e_pad = NTILE * chunks * CH

    blk = NTILE * WB
    rows_pad = -(-nu // blk) * blk

    tu0, tu1, tv0, tv1 = _project(x_u, x_v, weights_u, weights_v)

    def padr(t):
        return jnp.pad(t, ((0, rows_pad - t.shape[0]), (0, 0)))

    tabs = jnp.stack([jnp.stack([padr(tv0), padr(tv1)]),
                      jnp.stack([padr(tu0), padr(tu1)])])

    spread = jnp.arange(e_pad - e, dtype=I32) % nu

    def rs(a, padv):
        return jnp.concatenate([a, padv]).reshape(NTILE, chunks, CH)

    ei0 = edge_index_0.astype(I32)
    ei1 = edge_index_1.astype(I32)
    row0, col0 = ei0[0], ei0[1]
    row1, col1 = ei1[0], ei1[1]

    rc = jnp.stack([
        jnp.stack([rs(row0, spread), rs(col0, spread)], axis=2),
        jnp.stack([rs(row1, spread), rs(col1, spread)], axis=2),
    ])
    zpad = jnp.zeros(e_pad - e, F32)
    vl = jnp.stack([rs(edge_val_0.astype(F32), zpad),
                    rs(edge_val_1.astype(F32), zpad)])

    agg = _make_sc_agg(rows_pad, half, chunks)
    out = agg(tabs, rc, vl)
    return (out[0, :nu], out[1, :nv])
